# baseline (device time: 15343 ns/iter reference)
import jax
import jax.numpy as jnp
from jax import lax
from jax.experimental import pallas as pl
from jax.experimental.pallas import tpu as pltpu

N_DEV = 4
WIRE_SCALE = 256.0


def kernel(x, router_W, route_idx, expert_W, shared_W):
    m, d = x.shape
    e_local, _, h_dim = expert_W.shape
    n_experts = router_W.shape[1]

    def body(x_ref, rw_ref, idx_ref, ew_ref, sw_ref, out_ref,
             ew8_ref, full_ref, half_ref, send_sems, recv_sems):
        me = lax.axis_index("i")
        left = (me - 1) % N_DEV
        right = (me + 1) % N_DEV
        opp = (me + 2) % N_DEV

        barrier_sem = pltpu.get_barrier_semaphore()
        for nbr in (left, right):
            pl.semaphore_signal(
                barrier_sem, inc=1,
                device_id=(nbr,), device_id_type=pl.DeviceIdType.MESH,
            )
        pl.semaphore_wait(barrier_sem, 2)

        ew8_ref[0] = (ew_ref[0] * WIRE_SCALE).astype(jnp.float8_e4m3fn)

        to_r = [
            pltpu.make_async_remote_copy(
                src_ref=ew8_ref.at[j], dst_ref=full_ref.at[0, j],
                send_sem=send_sems.at[j], recv_sem=recv_sems.at[j],
                device_id=(right,), device_id_type=pl.DeviceIdType.MESH,
            )
            for j in range(2)
        ]
        to_l = [
            pltpu.make_async_remote_copy(
                src_ref=ew8_ref.at[j], dst_ref=full_ref.at[1, j],
                send_sem=send_sems.at[2 + j], recv_sem=recv_sems.at[2 + j],
                device_id=(left,), device_id_type=pl.DeviceIdType.MESH,
            )
            for j in range(2)
        ]
        fwd_r = pltpu.make_async_remote_copy(
            src_ref=full_ref.at[0, 0], dst_ref=half_ref.at[0],
            send_sem=send_sems.at[4], recv_sem=recv_sems.at[4],
            device_id=(right,), device_id_type=pl.DeviceIdType.MESH,
        )
        fwd_l = pltpu.make_async_remote_copy(
            src_ref=full_ref.at[1, 1], dst_ref=half_ref.at[1],
            send_sem=send_sems.at[5], recv_sem=recv_sems.at[5],
            device_id=(left,), device_id_type=pl.DeviceIdType.MESH,
        )
        to_r[0].start()
        ew8_ref[1] = (ew_ref[1] * WIRE_SCALE).astype(jnp.float8_e4m3fn)
        to_l[1].start()
        to_r[1].start()
        to_l[0].start()

        xv = x_ref[...]
        xh = xv.astype(jnp.bfloat16)
        ridx = idx_ref[...]

        scores = jnp.dot(xv, rw_ref[...], preferred_element_type=jnp.float32)
        s_max = jnp.max(scores, axis=1, keepdims=True)
        p = jnp.exp(scores - s_max)
        probs = p / jnp.sum(p, axis=1, keepdims=True)
        col = lax.broadcasted_iota(jnp.int32, (m, n_experts), 1)
        gate = jnp.sum(jnp.where(col == ridx, probs, 0.0),
                       axis=1, keepdims=True)
        gate8 = gate * (1.0 / WIRE_SCALE)

        def accum1(acc, w8, e, g=None):
            coeff = jnp.where(ridx == e, gate8 if g is None else g,
                              0.0).astype(jnp.bfloat16)
            return acc + jnp.dot(coeff * xh, w8.astype(jnp.bfloat16),
                                 preferred_element_type=jnp.float32)

        acc = jnp.dot(xh, sw_ref[...].astype(jnp.bfloat16),
                      preferred_element_type=jnp.float32)

        to_r[0].wait_recv()
        fwd_r.start()
        to_l[1].wait_recv()
        fwd_l.start()

        acc = accum1(acc, full_ref[0, 0], left * 2)
        acc = accum1(acc, full_ref[1, 1], right * 2 + 1)
        acc = accum1(acc, ew_ref[0], me * 2, gate)
        acc = accum1(acc, ew_ref[1], me * 2 + 1, gate)

        to_r[1].wait_recv()
        acc = accum1(acc, full_ref[0, 1], left * 2 + 1)
        to_l[0].wait_recv()
        acc = accum1(acc, full_ref[1, 0], right * 2)

        fwd_r.wait_recv()
        acc = accum1(acc, half_ref[0], opp * 2)
        fwd_l.wait_recv()
        acc = accum1(acc, half_ref[1], opp * 2 + 1)

        for rdma in (*to_r, *to_l, fwd_r, fwd_l):
            rdma.wait_send()
        out_ref[...] = acc

    return pl.pallas_call(
        body,
        out_shape=jax.ShapeDtypeStruct((m, h_dim), jnp.float32),
        in_specs=[pl.BlockSpec(memory_space=pltpu.VMEM)] * 5,
        out_specs=pl.BlockSpec(memory_space=pltpu.VMEM),
        scratch_shapes=[
            pltpu.VMEM((2, d, h_dim), jnp.float8_e4m3fn),
            pltpu.VMEM((2, e_local, d, h_dim), jnp.float8_e4m3fn),
            pltpu.VMEM((2, d, h_dim), jnp.float8_e4m3fn),
            pltpu.SemaphoreType.DMA((6,)),
            pltpu.SemaphoreType.DMA((6,)),
        ],
        compiler_params=pltpu.CompilerParams(collective_id=0),
    )(x, router_W, route_idx, expert_W, shared_W)
